# baseline (speedup 1.0000x reference)
import functools

import jax
import jax.numpy as jnp
from jax import lax
from jax.experimental import pallas as pl
from jax.experimental.pallas import tpu as pltpu
from jax.experimental.pallas import tpu_sc as plsc

L = 16
NW = 16
N_IN = 16384
CPT = N_IN // NW
NV = 5
D = 14

_mesh = plsc.VectorSubcoreMesh(core_axis_name="c", subcore_axis_name="s",
                               num_cores=1)


@functools.partial(
    pl.kernel,
    mesh=_mesh,
    out_type=jax.ShapeDtypeStruct((NW, L), jnp.float32),
    scratch_types=[
        pltpu.VMEM((CPT,), jnp.int32),
        pltpu.VMEM((L,), jnp.float32),
    ],
)
def _sc_histogram(e_hbm, out_hbm, idx_v, part_v):
    s = lax.axis_index("s")
    wid = s
    lane = lax.broadcasted_iota(jnp.int32, (L,), 0)

    def lane_sum(x):
        for sh in (8, 4, 2, 1):
            x = x + x.at[lane ^ sh].get(mode="promise_in_bounds",
                                        unique_indices=True)
        return x

    pltpu.sync_copy(e_hbm.at[pl.ds(wid * CPT, CPT)], idx_v)

    VPI = 1

    def body(it, acc):
        base = it * (VPI * L)
        for k in range(VPI):
            x = jnp.maximum(idx_v[pl.ds(base + k * L, L)], 0)
            acc = tuple(acc[v] + jnp.where(x == v, 1.0, 0.0)
                        for v in range(NV - 1))
        return acc

    acc = lax.fori_loop(0, CPT // (VPI * L), body,
                        tuple(jnp.zeros((L,), jnp.float32)
                              for _ in range(NV - 1)))
    cs = [lane_sum(a) for a in acc]
    cs.append(jnp.float32(CPT) - cs[0] - cs[1] - cs[2] - cs[3])
    part = jnp.zeros((L,), jnp.float32)
    for v in range(NV):
        part = jnp.where(lane == v, cs[v], part)
    part_v[...] = part
    pltpu.sync_copy(part_v, out_hbm.at[wid])


def _tc_tail(part_ref, a_ref, b_ref, w0_ref, w1_ref, w2_ref, out_ref):
    bf = jnp.bfloat16
    counts = jnp.sum(part_ref[...], axis=0)
    c5 = counts[:NV]
    e0 = jnp.sum(jnp.sum(w0_ref[...], axis=0) * c5[:, None], axis=0)
    e1 = jnp.sum(jnp.sum(w1_ref[...], axis=0) * c5[:, None], axis=0)
    e2 = jnp.sum(jnp.sum(w2_ref[...], axis=0) * c5[:, None], axis=0)
    mm0 = jax.lax.dot_general(
        a_ref[...].astype(bf), b_ref[...].astype(bf),
        (((1,), (0,)), ((), ())),
        preferred_element_type=jnp.float32)
    mm1 = mm0 * e0[None, :]
    mm2 = jax.lax.dot_general(
        e1[None, :].astype(bf), mm1.astype(bf),
        (((1,), (0,)), ((), ())),
        preferred_element_type=jnp.float32)
    s = jnp.sum(e2 * mm2[0, :])
    out_ref[...] = jnp.full((1, 1), s, jnp.float32)


_tc_tail_call = pl.pallas_call(
    _tc_tail,
    out_shape=jax.ShapeDtypeStruct((1, 1), jnp.float32),
)


def kernel(mm_0_a, mm_0_b, eb_input, eb_offset, W0, W1, W2):
    del eb_offset
    part = _sc_histogram(eb_input)
    return _tc_tail_call(part, mm_0_a, mm_0_b, W0, W1, W2)


# speedup vs baseline: 1.0003x; 1.0003x over previous
import functools

import jax
import jax.numpy as jnp
from jax import lax
from jax.experimental import pallas as pl
from jax.experimental.pallas import tpu as pltpu
from jax.experimental.pallas import tpu_sc as plsc

L = 16
NW = 16
N_IN = 16384
CPT = N_IN // NW
NV = 5
D = 14

_mesh = plsc.VectorSubcoreMesh(core_axis_name="c", subcore_axis_name="s",
                               num_cores=1)


@functools.partial(
    pl.kernel,
    mesh=_mesh,
    out_type=jax.ShapeDtypeStruct((NW, L), jnp.float32),
    scratch_types=[
        pltpu.VMEM((CPT,), jnp.int32),
        pltpu.VMEM((L,), jnp.float32),
    ],
)
def _sc_histogram(e_hbm, out_hbm, idx_v, part_v):
    s = lax.axis_index("s")
    wid = s
    lane = lax.broadcasted_iota(jnp.int32, (L,), 0)

    def lane_sum(x):
        for sh in (8, 4, 2, 1):
            x = x + x.at[lane ^ sh].get(mode="promise_in_bounds",
                                        unique_indices=True)
        return x

    pltpu.sync_copy(e_hbm.at[pl.ds(wid * CPT, CPT)], idx_v)

    VPI = 2

    def body(it, acc):
        base = it * (VPI * L)
        for k in range(VPI):
            x = jnp.maximum(idx_v[pl.ds(base + k * L, L)], 0)
            acc = tuple(acc[v] + jnp.where(x == v, 1.0, 0.0)
                        for v in range(NV - 1))
        return acc

    acc = lax.fori_loop(0, CPT // (VPI * L), body,
                        tuple(jnp.zeros((L,), jnp.float32)
                              for _ in range(NV - 1)))
    cs = [lane_sum(a) for a in acc]
    cs.append(jnp.float32(CPT) - cs[0] - cs[1] - cs[2] - cs[3])
    part = jnp.zeros((L,), jnp.float32)
    for v in range(NV):
        part = jnp.where(lane == v, cs[v], part)
    part_v[...] = part
    pltpu.sync_copy(part_v, out_hbm.at[wid])


def _tc_tail(part_ref, a_ref, b_ref, w0_ref, w1_ref, w2_ref, out_ref):
    bf = jnp.bfloat16
    counts = jnp.sum(part_ref[...], axis=0)
    c5 = counts[:NV]
    e0 = jnp.sum(jnp.sum(w0_ref[...], axis=0) * c5[:, None], axis=0)
    e1 = jnp.sum(jnp.sum(w1_ref[...], axis=0) * c5[:, None], axis=0)
    e2 = jnp.sum(jnp.sum(w2_ref[...], axis=0) * c5[:, None], axis=0)
    mm0 = jax.lax.dot_general(
        a_ref[...].astype(bf), b_ref[...].astype(bf),
        (((1,), (0,)), ((), ())),
        preferred_element_type=jnp.float32)
    mm1 = mm0 * e0[None, :]
    mm2 = jax.lax.dot_general(
        e1[None, :].astype(bf), mm1.astype(bf),
        (((1,), (0,)), ((), ())),
        preferred_element_type=jnp.float32)
    s = jnp.sum(e2 * mm2[0, :])
    out_ref[...] = jnp.full((1, 1), s, jnp.float32)


_tc_tail_call = pl.pallas_call(
    _tc_tail,
    out_shape=jax.ShapeDtypeStruct((1, 1), jnp.float32),
)


def kernel(mm_0_a, mm_0_b, eb_input, eb_offset, W0, W1, W2):
    del eb_offset
    part = _sc_histogram(eb_input)
    return _tc_tail_call(part, mm_0_a, mm_0_b, W0, W1, W2)
